# baseline (speedup 1.0000x reference)
import functools

import jax
import jax.numpy as jnp
from jax import lax
from jax.experimental import pallas as pl
from jax.experimental.pallas import tpu as pltpu
from jax.experimental.pallas import tpu_sc as plsc

N = 10000
D = 128

NC = 2
NS = 16
NW = NC * NS
CHUNK = 64
RING = 4
NCHE = 160
K0 = 160
K1 = 160
TOT_CH = NS * (K0 + K1)
E_PAD = TOT_CH * CHUNK
N_PAD = 10240
ROWS_SUB = N_PAD // NS
ROW_CH = ROWS_SUB // CHUNK

_MESH = plsc.VectorSubcoreMesh(
    core_axis_name="c", subcore_axis_name="s", num_cores=NC, num_subcores=NS
)


WCH = 64


def _sc_agg_body(h_hbm, src_hbm, dst_hbm, zeros_hbm, agg_out,
                 agg_s, rows_v, *rest):
    srcc = list(rest[0:RING])
    dstc = list(rest[RING:2 * RING])
    gsem = list(rest[2 * RING:3 * RING])
    ssem = list(rest[3 * RING:4 * RING])
    c = lax.axis_index("c")
    s = lax.axis_index("s")
    q0 = jnp.where(c == 0, s * K0, NS * K0 + s * K1)
    kc = jnp.where(c == 0, K0, K1)

    pltpu.sync_copy(zeros_hbm, rows_v.at[0, pl.ds(0, WCH)])
    base = s * ROWS_SUB
    for t in range(ROW_CH):
        pltpu.sync_copy(rows_v.at[0, pl.ds(0, WCH)],
                        agg_s.at[pl.ds(base + t * WCH, WCH)])
    for r in range(RING - 1):
        pltpu.sync_copy(src_hbm.at[q0 + r], srcc[r])
        pltpu.sync_copy(dst_hbm.at[q0 + r], dstc[r])
    plsc.subcore_barrier()
    for r in range(RING - 1):
        pltpu.async_copy(h_hbm.at[srcc[r]], rows_v.at[r, pl.ds(0, CHUNK)],
                         gsem[r])

    def ring_body(i, carry):
        for b in range(RING):
            j = RING * i + b
            nb = (b + RING - 1) % RING

            @pl.when(j >= 1)
            def _():
                pltpu.make_async_copy(zeros_hbm,
                                      rows_v.at[nb, pl.ds(0, CHUNK)],
                                      ssem[nb]).wait()

            @pl.when(j + RING - 1 < kc)
            def _():
                pltpu.sync_copy(src_hbm.at[q0 + j + RING - 1], srcc[nb])
                pltpu.sync_copy(dst_hbm.at[q0 + j + RING - 1], dstc[nb])
                pltpu.async_copy(h_hbm.at[srcc[nb]],
                                 rows_v.at[nb, pl.ds(0, CHUNK)], gsem[nb])

            pltpu.make_async_copy(zeros_hbm, rows_v.at[b, pl.ds(0, CHUNK)],
                                  gsem[b]).wait()
            pltpu.async_copy(rows_v.at[b, pl.ds(0, CHUNK)],
                             agg_s.at[dstc[b]], ssem[b], add=True)
        return carry

    lax.fori_loop(0, kc // RING, ring_body, 0)
    pltpu.make_async_copy(zeros_hbm, rows_v.at[(RING - 1) % RING,
                                               pl.ds(0, CHUNK)],
                          ssem[(RING - 1) % RING]).wait()
    plsc.subcore_barrier()

    for t in range(ROW_CH):
        r = base + t * WCH
        pltpu.sync_copy(agg_s.at[pl.ds(r, WCH)], rows_v.at[0, pl.ds(0, WCH)])
        pltpu.sync_copy(rows_v.at[0, pl.ds(0, WCH)],
                        agg_out.at[c, pl.ds(r, WCH)])


_sc_agg = pl.kernel(
    _sc_agg_body,
    out_type=jax.ShapeDtypeStruct((NC, N_PAD, D), jnp.float32),
    mesh=_MESH,
    scratch_types=[
        pltpu.VMEM_SHARED((N_PAD, D), jnp.float32),
        pltpu.VMEM((RING, CHUNK, D), jnp.float32),
    ] + [pltpu.VMEM((CHUNK,), jnp.int32)] * (2 * RING)
      + [pltpu.SemaphoreType.DMA] * (2 * RING),
    name="sage_sc_agg",
)


COLS_SUB = N_PAD // NS


def _sc_deg_body(dst_hbm, deg_out, hists_s, hist_v, dstv_v, stage_v, comb_v):
    c = lax.axis_index("c")
    s = lax.axis_index("s")
    wid = s * NC + c

    zero16 = jnp.zeros((16,), jnp.float32)

    def zbody(i, carry):
        hist_v[pl.ds(i * 16, 16)] = zero16
        return carry

    lax.fori_loop(0, N_PAD // 16, zbody, 0)
    pltpu.sync_copy(dst_hbm.at[pl.ds(wid * NCHE, NCHE)], dstv_v)

    ones = jnp.ones((16,), jnp.float32)

    gpr = CHUNK // 16

    def hbody(e, carry):
        idx = dstv_v[e // gpr, pl.ds((e % gpr) * 16, 16)]
        plsc.addupdate_scatter(hist_v, [idx], ones)
        return carry

    lax.fori_loop(0, (NCHE * CHUNK) // 16, hbody, 0)

    pltpu.sync_copy(hist_v, hists_s.at[s])
    plsc.subcore_barrier()

    cols = s * COLS_SUB
    pltpu.sync_copy(hists_s.at[:, pl.ds(cols, COLS_SUB)], stage_v)

    def cbody(k, carry):
        o = k * 16
        acc = stage_v[0, pl.ds(o, 16)]
        for r in range(1, NS):
            acc = acc + stage_v[r, pl.ds(o, 16)]
        comb_v[pl.ds(o, 16)] = acc
        return carry

    lax.fori_loop(0, COLS_SUB // 16, cbody, 0)
    pltpu.sync_copy(comb_v, deg_out.at[c, pl.ds(cols, COLS_SUB)])


_sc_deg = pl.kernel(
    _sc_deg_body,
    out_type=jax.ShapeDtypeStruct((NC, N_PAD), jnp.float32),
    mesh=_MESH,
    scratch_types=[
        pltpu.VMEM_SHARED((NS, N_PAD), jnp.float32),
        pltpu.VMEM((N_PAD,), jnp.float32),
        pltpu.VMEM((NCHE, CHUNK), jnp.int32),
        pltpu.VMEM((NS, COLS_SUB), jnp.float32),
        pltpu.VMEM((COLS_SUB,), jnp.float32),
    ],
    compiler_params=pltpu.CompilerParams(needs_layout_passes=False),
    name="sage_sc_deg",
)


R = 1024


def _tc_body(relu, x_ref, a0_ref, a1_ref, d0_ref, d1_ref, ws_ref, wn_ref,
             b_ref, o_ref):
    deg = jnp.maximum(d0_ref[...] + d1_ref[...], 1.0)
    hn = (a0_ref[...] + a1_ref[...]) / deg
    y = (
        jnp.dot(x_ref[...], ws_ref[...], preferred_element_type=jnp.float32)
        + jnp.dot(hn, wn_ref[...], preferred_element_type=jnp.float32)
        + b_ref[...]
    )
    o_ref[...] = jnp.maximum(y, 0.0) if relu else y


def _tc_layer(x_pad, a0, a1, d0, d1, WsT, WnT, b, relu):
    body = functools.partial(_tc_body, relu)
    grid = N_PAD // R
    row_spec = pl.BlockSpec((R, D), lambda i: (i, 0))
    deg_spec = pl.BlockSpec((R, 1), lambda i: (i, 0))
    w_spec = pl.BlockSpec((D, D), lambda i: (0, 0))
    b_spec = pl.BlockSpec((1, D), lambda i: (0, 0))
    return pl.pallas_call(
        body,
        grid=(grid,),
        in_specs=[row_spec, row_spec, row_spec, deg_spec, deg_spec,
                  w_spec, w_spec, b_spec],
        out_specs=row_spec,
        out_shape=jax.ShapeDtypeStruct((N_PAD, D), jnp.float32),
    )(x_pad, a0, a1, d0, d1, WsT, WnT, b.reshape(1, D))


def kernel(x, edge_index, Ws1, Wn1, b1, Ws2, Wn2, b2):
    src = edge_index[0]
    dst = edge_index[1]
    E = src.shape[0]
    pad = E_PAD - E
    src_p = jnp.concatenate([src, jnp.zeros((pad,), jnp.int32)])
    dst_p = jnp.concatenate([dst, jnp.full((pad,), N, jnp.int32)])
    src3 = src_p.reshape(TOT_CH, CHUNK)
    dst3 = dst_p.reshape(TOT_CH, CHUNK)

    x_pad = jnp.zeros((N_PAD, D), jnp.float32).at[:N].set(x)
    zeros_rows = jnp.zeros((CHUNK, D), jnp.float32)

    deg = _sc_deg(dst3)
    d0 = deg[0].reshape(N_PAD, 1)
    d1 = deg[1].reshape(N_PAD, 1)
    agg1 = _sc_agg(x_pad, src3, dst3, zeros_rows)
    h1 = _tc_layer(x_pad, agg1[0], agg1[1], d0, d1,
                   Ws1.T, Wn1.T, b1, relu=True)
    agg2 = _sc_agg(h1, src3, dst3, zeros_rows)
    out = _tc_layer(h1, agg2[0], agg2[1], d0, d1,
                    Ws2.T, Wn2.T, b2, relu=False)
    return out[:N]


# speedup vs baseline: 1.0971x; 1.0971x over previous
import functools

import jax
import jax.numpy as jnp
from jax import lax
from jax.experimental import pallas as pl
from jax.experimental.pallas import tpu as pltpu
from jax.experimental.pallas import tpu_sc as plsc

N = 10000
D = 128

NC = 2
NS = 16
NW = NC * NS
CHUNK = 64
RING = 4
NCHE = 160
K0 = 236
K1 = 84
TOT_CH = NS * (K0 + K1)
E_PAD = TOT_CH * CHUNK
N_PAD = 10240
ROWS_SUB = N_PAD // NS
ROW_CH = ROWS_SUB // CHUNK

_MESH = plsc.VectorSubcoreMesh(
    core_axis_name="c", subcore_axis_name="s", num_cores=NC, num_subcores=NS
)


WCH = 64


def _sc_agg_body(h_hbm, src_hbm, dst_hbm, zeros_hbm, agg_out,
                 agg_s, rows_v, *rest):
    srcc = list(rest[0:RING])
    dstc = list(rest[RING:2 * RING])
    gsem = list(rest[2 * RING:3 * RING])
    ssem = list(rest[3 * RING:4 * RING])
    c = lax.axis_index("c")
    s = lax.axis_index("s")
    q0 = jnp.where(c == 0, s * K0, NS * K0 + s * K1)
    kc = jnp.where(c == 0, K0, K1)

    pltpu.sync_copy(zeros_hbm, rows_v.at[0, pl.ds(0, WCH)])
    base = s * ROWS_SUB
    for t in range(ROW_CH):
        pltpu.sync_copy(rows_v.at[0, pl.ds(0, WCH)],
                        agg_s.at[pl.ds(base + t * WCH, WCH)])
    for r in range(RING - 1):
        pltpu.sync_copy(src_hbm.at[q0 + r], srcc[r])
        pltpu.sync_copy(dst_hbm.at[q0 + r], dstc[r])
    plsc.subcore_barrier()
    for r in range(RING - 1):
        pltpu.async_copy(h_hbm.at[srcc[r]], rows_v.at[r, pl.ds(0, CHUNK)],
                         gsem[r])

    def ring_body(i, carry):
        for b in range(RING):
            j = RING * i + b
            nb = (b + RING - 1) % RING

            @pl.when(j >= 1)
            def _():
                pltpu.make_async_copy(zeros_hbm,
                                      rows_v.at[nb, pl.ds(0, CHUNK)],
                                      ssem[nb]).wait()

            @pl.when(j + RING - 1 < kc)
            def _():
                pltpu.sync_copy(src_hbm.at[q0 + j + RING - 1], srcc[nb])
                pltpu.sync_copy(dst_hbm.at[q0 + j + RING - 1], dstc[nb])
                pltpu.async_copy(h_hbm.at[srcc[nb]],
                                 rows_v.at[nb, pl.ds(0, CHUNK)], gsem[nb])

            pltpu.make_async_copy(zeros_hbm, rows_v.at[b, pl.ds(0, CHUNK)],
                                  gsem[b]).wait()
            pltpu.async_copy(rows_v.at[b, pl.ds(0, CHUNK)],
                             agg_s.at[dstc[b]], ssem[b], add=True)
        return carry

    lax.fori_loop(0, kc // RING, ring_body, 0)
    pltpu.make_async_copy(zeros_hbm, rows_v.at[(RING - 1) % RING,
                                               pl.ds(0, CHUNK)],
                          ssem[(RING - 1) % RING]).wait()
    plsc.subcore_barrier()

    for t in range(ROW_CH):
        r = base + t * WCH
        pltpu.sync_copy(agg_s.at[pl.ds(r, WCH)], rows_v.at[0, pl.ds(0, WCH)])
        pltpu.sync_copy(rows_v.at[0, pl.ds(0, WCH)],
                        agg_out.at[c, pl.ds(r, WCH)])


_sc_agg = pl.kernel(
    _sc_agg_body,
    out_type=jax.ShapeDtypeStruct((NC, N_PAD, D), jnp.float32),
    mesh=_MESH,
    scratch_types=[
        pltpu.VMEM_SHARED((N_PAD, D), jnp.float32),
        pltpu.VMEM((RING, CHUNK, D), jnp.float32),
    ] + [pltpu.VMEM((CHUNK,), jnp.int32)] * (2 * RING)
      + [pltpu.SemaphoreType.DMA] * (2 * RING),
    name="sage_sc_agg",
)


COLS_SUB = N_PAD // NS


def _sc_deg_body(dst_hbm, deg_out, hists_s, hist_v, dstv_v, stage_v, comb_v):
    c = lax.axis_index("c")
    s = lax.axis_index("s")
    wid = s * NC + c

    zero16 = jnp.zeros((16,), jnp.float32)

    def zbody(i, carry):
        hist_v[pl.ds(i * 16, 16)] = zero16
        return carry

    lax.fori_loop(0, N_PAD // 16, zbody, 0)
    pltpu.sync_copy(dst_hbm.at[pl.ds(wid * NCHE, NCHE)], dstv_v)

    ones = jnp.ones((16,), jnp.float32)

    gpr = CHUNK // 16

    def hbody(e, carry):
        idx = dstv_v[e // gpr, pl.ds((e % gpr) * 16, 16)]
        plsc.addupdate_scatter(hist_v, [idx], ones)
        return carry

    lax.fori_loop(0, (NCHE * CHUNK) // 16, hbody, 0)

    pltpu.sync_copy(hist_v, hists_s.at[s])
    plsc.subcore_barrier()

    cols = s * COLS_SUB
    pltpu.sync_copy(hists_s.at[:, pl.ds(cols, COLS_SUB)], stage_v)

    def cbody(k, carry):
        o = k * 16
        acc = stage_v[0, pl.ds(o, 16)]
        for r in range(1, NS):
            acc = acc + stage_v[r, pl.ds(o, 16)]
        comb_v[pl.ds(o, 16)] = acc
        return carry

    lax.fori_loop(0, COLS_SUB // 16, cbody, 0)
    pltpu.sync_copy(comb_v, deg_out.at[c, pl.ds(cols, COLS_SUB)])


_sc_deg = pl.kernel(
    _sc_deg_body,
    out_type=jax.ShapeDtypeStruct((NC, N_PAD), jnp.float32),
    mesh=_MESH,
    scratch_types=[
        pltpu.VMEM_SHARED((NS, N_PAD), jnp.float32),
        pltpu.VMEM((N_PAD,), jnp.float32),
        pltpu.VMEM((NCHE, CHUNK), jnp.int32),
        pltpu.VMEM((NS, COLS_SUB), jnp.float32),
        pltpu.VMEM((COLS_SUB,), jnp.float32),
    ],
    compiler_params=pltpu.CompilerParams(needs_layout_passes=False),
    name="sage_sc_deg",
)


R = 1024


def _tc_body(relu, x_ref, a0_ref, a1_ref, d0_ref, d1_ref, ws_ref, wn_ref,
             b_ref, o_ref):
    deg = jnp.maximum(d0_ref[...] + d1_ref[...], 1.0)
    hn = (a0_ref[...] + a1_ref[...]) / deg
    y = (
        jnp.dot(x_ref[...], ws_ref[...], preferred_element_type=jnp.float32)
        + jnp.dot(hn, wn_ref[...], preferred_element_type=jnp.float32)
        + b_ref[...]
    )
    o_ref[...] = jnp.maximum(y, 0.0) if relu else y


def _tc_layer(x_pad, a0, a1, d0, d1, WsT, WnT, b, relu):
    body = functools.partial(_tc_body, relu)
    grid = N_PAD // R
    row_spec = pl.BlockSpec((R, D), lambda i: (i, 0))
    deg_spec = pl.BlockSpec((R, 1), lambda i: (i, 0))
    w_spec = pl.BlockSpec((D, D), lambda i: (0, 0))
    b_spec = pl.BlockSpec((1, D), lambda i: (0, 0))
    return pl.pallas_call(
        body,
        grid=(grid,),
        in_specs=[row_spec, row_spec, row_spec, deg_spec, deg_spec,
                  w_spec, w_spec, b_spec],
        out_specs=row_spec,
        out_shape=jax.ShapeDtypeStruct((N_PAD, D), jnp.float32),
    )(x_pad, a0, a1, d0, d1, WsT, WnT, b.reshape(1, D))


def kernel(x, edge_index, Ws1, Wn1, b1, Ws2, Wn2, b2):
    src = edge_index[0]
    dst = edge_index[1]
    E = src.shape[0]
    pad = E_PAD - E
    src_p = jnp.concatenate([src, jnp.zeros((pad,), jnp.int32)])
    dst_p = jnp.concatenate([dst, jnp.full((pad,), N, jnp.int32)])
    src3 = src_p.reshape(TOT_CH, CHUNK)
    dst3 = dst_p.reshape(TOT_CH, CHUNK)

    x_pad = jnp.zeros((N_PAD, D), jnp.float32).at[:N].set(x)
    zeros_rows = jnp.zeros((CHUNK, D), jnp.float32)

    deg = _sc_deg(dst3)
    d0 = deg[0].reshape(N_PAD, 1)
    d1 = deg[1].reshape(N_PAD, 1)
    agg1 = _sc_agg(x_pad, src3, dst3, zeros_rows)
    h1 = _tc_layer(x_pad, agg1[0], agg1[1], d0, d1,
                   Ws1.T, Wn1.T, b1, relu=True)
    agg2 = _sc_agg(h1, src3, dst3, zeros_rows)
    out = _tc_layer(h1, agg2[0], agg2[1], d0, d1,
                    Ws2.T, Wn2.T, b2, relu=False)
    return out[:N]
